# baseline (speedup 1.0000x reference)
import functools

import jax
import jax.numpy as jnp
from jax.experimental import pallas as pl
from jax.experimental.pallas import tpu as pltpu


def _xw_kernel(x_ref, w_ref, y_ref):
    y_ref[...] = jnp.dot(x_ref[...], w_ref[...],
                         preferred_element_type=jnp.float32).astype(y_ref.dtype)


def _agg_kernel(x_ref, wr_ref, a_ref, y_ref, o_ref, *, tk):
    k = pl.program_id(1)

    @pl.when(k == 0)
    def _init():
        o_ref[...] = jnp.dot(x_ref[...], wr_ref[...],
                             preferred_element_type=jnp.float32)

    start = pl.multiple_of(k * tk, tk)
    y_blk = y_ref[pl.ds(start, tk), :]
    o_ref[...] += jnp.dot(a_ref[...], y_blk, preferred_element_type=jnp.float32)


def kernel(x, edge_index, weight, weight_r):
    N, f_in = x.shape
    f_out = weight.shape[1]

    row = edge_index[0]
    col = edge_index[1]
    deg = jnp.zeros((N,), jnp.float32).at[col].add(1.0)
    rd = jnp.where(deg > 0, jax.lax.rsqrt(deg), 0.0)
    value = (rd[col] * rd[row]).astype(jnp.bfloat16)
    adj = jnp.zeros((N, N), jnp.bfloat16).at[col, row].add(value)

    xb = x.astype(jnp.bfloat16)
    wb = weight.astype(jnp.bfloat16)
    wrb = weight_r.astype(jnp.bfloat16)

    TM = TK = 1024

    y = pl.pallas_call(
        _xw_kernel,
        out_shape=jax.ShapeDtypeStruct((N, f_out), jnp.bfloat16),
        grid=(N // TM,),
        in_specs=[
            pl.BlockSpec((TM, f_in), lambda i: (i, 0)),
            pl.BlockSpec((f_in, f_out), lambda i: (0, 0)),
        ],
        out_specs=pl.BlockSpec((TM, f_out), lambda i: (i, 0)),
        compiler_params=pltpu.CompilerParams(
            dimension_semantics=("parallel",),
            vmem_limit_bytes=32 * 1024 * 1024),
    )(xb, wb)

    out = pl.pallas_call(
        functools.partial(_agg_kernel, tk=TK),
        out_shape=jax.ShapeDtypeStruct((N, f_out), jnp.float32),
        grid=(N // TM, N // TK),
        in_specs=[
            pl.BlockSpec((TM, f_in), lambda i, k: (i, 0)),
            pl.BlockSpec((f_in, f_out), lambda i, k: (0, 0)),
            pl.BlockSpec((TM, TK), lambda i, k: (i, k)),
            pl.BlockSpec((N, f_out), lambda i, k: (0, 0)),
        ],
        out_specs=pl.BlockSpec((TM, f_out), lambda i, k: (i, 0)),
        compiler_params=pltpu.CompilerParams(
            dimension_semantics=("parallel", "arbitrary"),
            vmem_limit_bytes=32 * 1024 * 1024),
    )(xb, wrb, adj, y)

    return out


# speedup vs baseline: 4.6423x; 4.6423x over previous
import functools

import jax
import jax.numpy as jnp
from jax.experimental import pallas as pl
from jax.experimental.pallas import tpu as pltpu


def _deg_kernel(bounds_ref, cols_ref, deg_ref, *, tm, u):
    i = pl.program_id(0)
    c0 = i * tm
    k_lo = bounds_ref[i] // u
    k_hi = (bounds_ref[i + 1] + u - 1) // u

    deg_ref[...] = jnp.zeros_like(deg_ref)
    col_iota = c0 + jax.lax.broadcasted_iota(jnp.int32, (tm, u), 0)

    def body(k, _):
        cvec = cols_ref[k, 0, :].reshape(1, u)
        pt = (col_iota == cvec).astype(jnp.float32)
        deg_ref[...] += jnp.sum(pt, axis=1, keepdims=True)
        return _

    jax.lax.fori_loop(k_lo, k_hi, body, None)


def _y_kernel(x_ref, w_ref, deg_ref, y_ref):
    y = jnp.dot(x_ref[...], w_ref[...], preferred_element_type=jnp.float32)
    deg = deg_ref[...]
    rd = jnp.where(deg > 0.0, jax.lax.rsqrt(deg), 0.0)
    y_ref[...] = y * rd


def _agg_kernel(bounds_ref, rows_ref, cols_ref, x_ref, wr_ref, y_ref, o_ref,
                tile_ref, deg_ref, *, tm, u, f_out):
    i = pl.program_id(0)
    c0 = i * tm
    k_lo = bounds_ref[i] // u
    k_hi = (bounds_ref[i + 1] + u - 1) // u

    o_ref[...] = jnp.zeros_like(o_ref)
    deg_ref[...] = jnp.zeros_like(deg_ref)
    col_iota = c0 + jax.lax.broadcasted_iota(jnp.int32, (tm, u), 0)

    def body(k, _):
        e0 = k * u
        for mi in range(u):
            r = rows_ref[e0 + mi]
            base = pl.multiple_of((r >> 3) << 3, 8)
            chunk = y_ref[pl.ds(base, 8), :]
            rowv = pltpu.roll(chunk, (8 - (r & 7)) & 7, axis=0)
            tile_ref[pl.ds(mi, 1), :] = rowv[0:1, :]

        cvec = cols_ref[k, 0, :].reshape(1, u)
        pt = (col_iota == cvec).astype(jnp.bfloat16)
        tile = tile_ref[...].astype(jnp.bfloat16)
        o_ref[...] += jnp.dot(pt, tile, preferred_element_type=jnp.float32)
        deg_ref[...] += jnp.sum(pt.astype(jnp.float32), axis=1, keepdims=True)
        return _

    jax.lax.fori_loop(k_lo, k_hi, body, None)

    deg = deg_ref[...]
    rdc = jnp.where(deg > 0.0, jax.lax.rsqrt(deg), 0.0)
    xwr = jnp.dot(x_ref[...], wr_ref[...], preferred_element_type=jnp.float32)
    o_ref[...] = xwr + rdc * o_ref[...]


def kernel(x, edge_index, weight, weight_r):
    N, f_in = x.shape
    f_out = weight.shape[1]
    E = edge_index.shape[1]

    TM = 256
    U = 256
    NBLK = N // TM

    row = edge_index[0]
    col = edge_index[1]
    shift = max(int(N - 1).bit_length(), 1)
    key = (col << shift) | row
    e_pad = ((E + U - 1) // U) * U
    if e_pad != E:
        key = jnp.concatenate(
            [key, jnp.full((e_pad - E,), N << shift, jnp.int32)])
    skey = jnp.sort(key)
    col_s = skey >> shift
    row_s = skey & (N - 1)
    bounds = jnp.searchsorted(
        col_s, jnp.arange(0, N + 1, TM, dtype=jnp.int32),
        side="left").astype(jnp.int32)
    cols3d = col_s.reshape(e_pad // U, 1, U)

    xb = x.astype(jnp.bfloat16)
    wb = weight.astype(jnp.bfloat16)
    wrb = weight_r.astype(jnp.bfloat16)

    deg = pl.pallas_call(
        functools.partial(_deg_kernel, tm=TM, u=U),
        out_shape=jax.ShapeDtypeStruct((N, 1), jnp.float32),
        grid_spec=pltpu.PrefetchScalarGridSpec(
            num_scalar_prefetch=1,
            grid=(NBLK,),
            in_specs=[
                pl.BlockSpec((e_pad // U, 1, U), lambda i, *_: (0, 0, 0)),
            ],
            out_specs=pl.BlockSpec((TM, 1), lambda i, *_: (i, 0)),
        ),
        compiler_params=pltpu.CompilerParams(
            dimension_semantics=("parallel",),
            vmem_limit_bytes=40 * 1024 * 1024),
    )(bounds, cols3d)

    TY = min(1024, N)
    yt = pl.pallas_call(
        _y_kernel,
        out_shape=jax.ShapeDtypeStruct((N, f_out), jnp.float32),
        grid=(N // TY,),
        in_specs=[
            pl.BlockSpec((TY, f_in), lambda i: (i, 0)),
            pl.BlockSpec((f_in, f_out), lambda i: (0, 0)),
            pl.BlockSpec((TY, 1), lambda i: (i, 0)),
        ],
        out_specs=pl.BlockSpec((TY, f_out), lambda i: (i, 0)),
        compiler_params=pltpu.CompilerParams(
            dimension_semantics=("parallel",),
            vmem_limit_bytes=40 * 1024 * 1024),
    )(xb, wb, deg)

    out = pl.pallas_call(
        functools.partial(_agg_kernel, tm=TM, u=U, f_out=f_out),
        out_shape=jax.ShapeDtypeStruct((N, f_out), jnp.float32),
        grid_spec=pltpu.PrefetchScalarGridSpec(
            num_scalar_prefetch=2,
            grid=(NBLK,),
            in_specs=[
                pl.BlockSpec((e_pad // U, 1, U), lambda i, *_: (0, 0, 0)),
                pl.BlockSpec((TM, f_in), lambda i, *_: (i, 0)),
                pl.BlockSpec((f_in, f_out), lambda i, *_: (0, 0)),
                pl.BlockSpec((N, f_out), lambda i, *_: (0, 0)),
            ],
            out_specs=pl.BlockSpec((TM, f_out), lambda i, *_: (i, 0)),
            scratch_shapes=[
                pltpu.VMEM((U, f_out), jnp.float32),
                pltpu.VMEM((TM, 1), jnp.float32),
            ],
        ),
        compiler_params=pltpu.CompilerParams(
            dimension_semantics=("parallel",),
            vmem_limit_bytes=40 * 1024 * 1024),
    )(bounds, row_s, cols3d, xb, wrb, yt)

    return out


# speedup vs baseline: 4.7334x; 1.0196x over previous
import functools

import jax
import jax.numpy as jnp
from jax.experimental import pallas as pl
from jax.experimental.pallas import tpu as pltpu


def _deg_kernel(bounds_ref, cols_ref, deg_ref, *, tm, u):
    i = pl.program_id(0)
    c0 = i * tm
    k_lo = bounds_ref[i] // u
    k_hi = (bounds_ref[i + 1] + u - 1) // u

    deg_ref[...] = jnp.zeros_like(deg_ref)
    col_iota = c0 + jax.lax.broadcasted_iota(jnp.int32, (tm, u), 0)

    def body(k, carry):
        cvec = cols_ref[k, 0, :].reshape(1, u)
        pt = (col_iota == cvec).astype(jnp.float32)
        deg_ref[...] += jnp.sum(pt, axis=1, keepdims=True)
        return carry

    jax.lax.fori_loop(k_lo, k_hi, body, None)


def _y_kernel(x_ref, w_ref, deg_ref, y_ref):
    y = jnp.dot(x_ref[...], w_ref[...], preferred_element_type=jnp.float32)
    deg = deg_ref[...]
    rd = jnp.where(deg > 0.0, jax.lax.rsqrt(deg), 0.0)
    y_ref[...] = (y * rd).astype(jnp.bfloat16)


def _agg_kernel(bounds_ref, rows_ref, cols_ref, x_ref, wr_ref, y_ref, o_ref,
                tile_ref, deg_ref, *, tm, u, f_out):
    i = pl.program_id(0)
    c0 = i * tm
    k_lo = bounds_ref[i] // u
    k_hi = (bounds_ref[i + 1] + u - 1) // u

    o_ref[...] = jnp.zeros_like(o_ref)
    deg_ref[...] = jnp.zeros_like(deg_ref)
    col_iota = c0 + jax.lax.broadcasted_iota(jnp.int32, (tm, u), 0)
    ones_u = jnp.ones((u, 8), jnp.bfloat16)
    half = f_out // 2

    def body(k, carry):
        e0 = k * u
        for mi in range(u):
            p = rows_ref[e0 + mi]
            base = pl.multiple_of(p & 65528, 8)
            sh = p >> 16
            chunk = y_ref[pl.ds(base, 8), :]
            rowv = pltpu.roll(chunk, sh, axis=0)
            tile_ref[pl.ds(mi, 1), :] = rowv[0:1, :]

        cvec = cols_ref[k, 0, :].reshape(1, u)
        pt = (col_iota == cvec).astype(jnp.bfloat16)

        t = tile_ref[...]
        even = jax.lax.bitcast_convert_type(t << 16, jnp.float32)
        odd = jax.lax.bitcast_convert_type(t & jnp.int32(-65536), jnp.float32)
        eo = jnp.concatenate(
            [even.astype(jnp.bfloat16), odd.astype(jnp.bfloat16)], axis=1)

        o_ref[...] += jnp.dot(pt, eo, preferred_element_type=jnp.float32)
        deg_ref[...] += jnp.dot(pt, ones_u, preferred_element_type=jnp.float32)
        return carry

    jax.lax.fori_loop(k_lo, k_hi, body, None)

    deg = deg_ref[:, 0:1]
    rdc = jnp.where(deg > 0.0, jax.lax.rsqrt(deg), 0.0)
    xwr = jnp.dot(x_ref[...], wr_ref[...], preferred_element_type=jnp.float32)
    o_ref[...] = xwr + rdc * o_ref[...]


def kernel(x, edge_index, weight, weight_r):
    N, f_in = x.shape
    f_out = weight.shape[1]
    E = edge_index.shape[1]

    TM = 256
    U = 256
    NBLK = N // TM
    half = f_out // 2

    row = edge_index[0]
    col = edge_index[1]
    shift = max(int(N - 1).bit_length(), 1)
    key = (col << shift) | row
    e_pad = ((E + U - 1) // U) * U
    if e_pad != E:
        key = jnp.concatenate(
            [key, jnp.full((e_pad - E,), N << shift, jnp.int32)])
    skey = jnp.sort(key)
    col_s = skey >> shift
    row_s = skey & (N - 1)
    rows_packed = row_s | (((8 - (row_s & 7)) & 7) << 16)
    bounds = jnp.searchsorted(
        col_s, jnp.arange(0, N + 1, TM, dtype=jnp.int32),
        side="left").astype(jnp.int32)
    cols3d = col_s.reshape(e_pad // U, 1, U)

    xb = x.astype(jnp.bfloat16)
    wb = weight.astype(jnp.bfloat16)
    perm = jnp.concatenate([jnp.arange(0, f_out, 2, dtype=jnp.int32),
                            jnp.arange(1, f_out, 2, dtype=jnp.int32)])
    wrp = weight_r.astype(jnp.bfloat16)[:, perm]

    deg = pl.pallas_call(
        functools.partial(_deg_kernel, tm=TM, u=U),
        out_shape=jax.ShapeDtypeStruct((N, 1), jnp.float32),
        grid_spec=pltpu.PrefetchScalarGridSpec(
            num_scalar_prefetch=1,
            grid=(NBLK,),
            in_specs=[
                pl.BlockSpec((e_pad // U, 1, U), lambda i, *_: (0, 0, 0)),
            ],
            out_specs=pl.BlockSpec((TM, 1), lambda i, *_: (i, 0)),
        ),
        compiler_params=pltpu.CompilerParams(
            dimension_semantics=("parallel",),
            vmem_limit_bytes=40 * 1024 * 1024),
    )(bounds, cols3d)

    TY = min(1024, N)
    yt = pl.pallas_call(
        _y_kernel,
        out_shape=jax.ShapeDtypeStruct((N, f_out), jnp.bfloat16),
        grid=(N // TY,),
        in_specs=[
            pl.BlockSpec((TY, f_in), lambda i: (i, 0)),
            pl.BlockSpec((f_in, f_out), lambda i: (0, 0)),
            pl.BlockSpec((TY, 1), lambda i: (i, 0)),
        ],
        out_specs=pl.BlockSpec((TY, f_out), lambda i: (i, 0)),
        compiler_params=pltpu.CompilerParams(
            dimension_semantics=("parallel",),
            vmem_limit_bytes=40 * 1024 * 1024),
    )(xb, wb, deg)

    y_i32 = jax.lax.bitcast_convert_type(
        yt.reshape(N, half, 2), jnp.int32)

    out_p = pl.pallas_call(
        functools.partial(_agg_kernel, tm=TM, u=U, f_out=f_out),
        out_shape=jax.ShapeDtypeStruct((N, f_out), jnp.float32),
        grid_spec=pltpu.PrefetchScalarGridSpec(
            num_scalar_prefetch=2,
            grid=(NBLK,),
            in_specs=[
                pl.BlockSpec((e_pad // U, 1, U), lambda i, *_: (0, 0, 0)),
                pl.BlockSpec((TM, f_in), lambda i, *_: (i, 0)),
                pl.BlockSpec((f_in, f_out), lambda i, *_: (0, 0)),
                pl.BlockSpec((N, half), lambda i, *_: (0, 0)),
            ],
            out_specs=pl.BlockSpec((TM, f_out), lambda i, *_: (i, 0)),
            scratch_shapes=[
                pltpu.VMEM((U, half), jnp.int32),
                pltpu.VMEM((TM, 8), jnp.float32),
            ],
        ),
        compiler_params=pltpu.CompilerParams(
            dimension_semantics=("parallel",),
            vmem_limit_bytes=40 * 1024 * 1024),
    )(bounds, rows_packed, cols3d, xb, wrp, y_i32)

    inv = jnp.stack([jnp.arange(half, dtype=jnp.int32),
                     half + jnp.arange(half, dtype=jnp.int32)],
                    axis=1).reshape(f_out)
    return out_p[:, inv]


# speedup vs baseline: 6.4794x; 1.3689x over previous
import functools

import jax
import jax.numpy as jnp
from jax.experimental import pallas as pl
from jax.experimental.pallas import tpu as pltpu


def _deg_kernel(cols_ref, deg_ref, *, u, nhi):
    k = pl.program_id(0)

    @pl.when(k == 0)
    def _init():
        deg_ref[...] = jnp.zeros_like(deg_ref)

    cvec = cols_ref[0, 0, :].reshape(1, u)
    lo_iota = jax.lax.broadcasted_iota(jnp.int32, (128, u), 0)
    hi_iota = jax.lax.broadcasted_iota(jnp.int32, (nhi, u), 0)
    xlo = (lo_iota == (cvec & 127)).astype(jnp.bfloat16)
    xhi = (hi_iota == (cvec >> 7)).astype(jnp.bfloat16)
    deg_ref[...] += jax.lax.dot_general(
        xlo, xhi, (((1,), (1,)), ((), ())),
        preferred_element_type=jnp.float32)


def _y_kernel(x_ref, w_ref, deg_ref, y_ref):
    xb = x_ref[...].astype(jnp.bfloat16)
    y = jnp.dot(xb, w_ref[...], preferred_element_type=jnp.float32)
    deg = deg_ref[...]
    rd = jnp.where(deg > 0.0, jax.lax.rsqrt(deg), 0.0)
    y_ref[...] = (y * rd).astype(jnp.bfloat16)


def _agg_kernel(bounds_ref, rows_ref, cols_ref, x_ref, wr_ref, deg_ref,
                y_ref, o_ref, tile_ref, *, tm, u, f_out):
    i = pl.program_id(0)
    c0 = i * tm
    k_lo = bounds_ref[i] // u
    k_hi = (bounds_ref[i + 1] + u - 1) // u

    o_ref[...] = jnp.zeros_like(o_ref)
    col_iota = c0 + jax.lax.broadcasted_iota(jnp.int32, (tm, u), 0)

    def body(k, carry):
        e0 = k * u
        for mi in range(u):
            p = rows_ref[e0 + mi]
            base = pl.multiple_of(p & 65528, 8)
            sh = p >> 16
            chunk = y_ref[pl.ds(base, 8), :]
            rowv = pltpu.roll(chunk, sh, axis=0)
            tile_ref[pl.ds(mi, 1), :] = rowv[0:1, :]

        cvec = cols_ref[k, 0, :].reshape(1, u)
        pt = (col_iota == cvec).astype(jnp.bfloat16)

        t = tile_ref[...]
        even = jax.lax.bitcast_convert_type(t << 16, jnp.float32)
        odd = jax.lax.bitcast_convert_type(t & jnp.int32(-65536), jnp.float32)
        eo = jnp.concatenate(
            [even.astype(jnp.bfloat16), odd.astype(jnp.bfloat16)], axis=1)

        o_ref[...] += jnp.dot(pt, eo, preferred_element_type=jnp.float32)
        return carry

    jax.lax.fori_loop(k_lo, k_hi, body, None)

    deg = deg_ref[...]
    rdc = jnp.where(deg > 0.0, jax.lax.rsqrt(deg), 0.0)
    xb = x_ref[...].astype(jnp.bfloat16)
    xwr = jnp.dot(xb, wr_ref[...], preferred_element_type=jnp.float32)
    o_ref[...] = xwr + rdc * o_ref[...]


def kernel(x, edge_index, weight, weight_r):
    N, f_in = x.shape
    f_out = weight.shape[1]
    E = edge_index.shape[1]

    TM = 256
    U = 512
    NBLK = N // TM
    half = f_out // 2

    row = edge_index[0]
    col = edge_index[1]
    shift = max(int(N - 1).bit_length(), 1)
    key = (col << shift) | row
    e_pad = ((E + U - 1) // U) * U
    if e_pad != E:
        key = jnp.concatenate(
            [key, jnp.full((e_pad - E,), N << shift, jnp.int32)])
    skey = jax.lax.sort(key, is_stable=False)
    col_s = skey >> shift
    row_s = skey & (N - 1)
    rows_packed = row_s | (((8 - (row_s & 7)) & 7) << 16)
    cols3d = col_s.reshape(e_pad // U, 1, U)

    q = jnp.stack([jnp.arange(half, dtype=jnp.int32),
                   half + jnp.arange(half, dtype=jnp.int32)],
                  axis=1).reshape(f_out)
    wq = weight.astype(jnp.bfloat16)[:, q]
    wrb = weight_r.astype(jnp.bfloat16)

    degT = pl.pallas_call(
        functools.partial(_deg_kernel, u=U, nhi=N // 128),
        out_shape=jax.ShapeDtypeStruct((128, N // 128), jnp.float32),
        grid=(e_pad // U,),
        in_specs=[
            pl.BlockSpec((1, 1, U), lambda k: (k, 0, 0)),
        ],
        out_specs=pl.BlockSpec((128, N // 128), lambda k: (0, 0)),
        compiler_params=pltpu.CompilerParams(
            dimension_semantics=("arbitrary",),
            vmem_limit_bytes=40 * 1024 * 1024),
    )(cols3d)

    deg_lin = degT.T.reshape(N)
    deg_col = deg_lin[:, None]
    bounds = jnp.concatenate([
        jnp.zeros((1,), jnp.int32),
        jnp.cumsum(deg_lin.reshape(NBLK, TM).sum(axis=1)).astype(jnp.int32),
    ])

    TY = min(1024, N)
    yt = pl.pallas_call(
        _y_kernel,
        out_shape=jax.ShapeDtypeStruct((N, f_out), jnp.bfloat16),
        grid=(N // TY,),
        in_specs=[
            pl.BlockSpec((TY, f_in), lambda i: (i, 0)),
            pl.BlockSpec((f_in, f_out), lambda i: (0, 0)),
            pl.BlockSpec((TY, 1), lambda i: (i, 0)),
        ],
        out_specs=pl.BlockSpec((TY, f_out), lambda i: (i, 0)),
        compiler_params=pltpu.CompilerParams(
            dimension_semantics=("parallel",),
            vmem_limit_bytes=40 * 1024 * 1024),
    )(x, wq, deg_col)

    y_i32 = jax.lax.bitcast_convert_type(
        yt.reshape(N, half, 2), jnp.int32)

    out = pl.pallas_call(
        functools.partial(_agg_kernel, tm=TM, u=U, f_out=f_out),
        out_shape=jax.ShapeDtypeStruct((N, f_out), jnp.float32),
        grid_spec=pltpu.PrefetchScalarGridSpec(
            num_scalar_prefetch=2,
            grid=(NBLK,),
            in_specs=[
                pl.BlockSpec((e_pad // U, 1, U), lambda i, *_: (0, 0, 0)),
                pl.BlockSpec((TM, f_in), lambda i, *_: (i, 0)),
                pl.BlockSpec((f_in, f_out), lambda i, *_: (0, 0)),
                pl.BlockSpec((TM, 1), lambda i, *_: (i, 0)),
                pl.BlockSpec((N, half), lambda i, *_: (0, 0)),
            ],
            out_specs=pl.BlockSpec((TM, f_out), lambda i, *_: (i, 0)),
            scratch_shapes=[
                pltpu.VMEM((U, half), jnp.int32),
            ],
        ),
        compiler_params=pltpu.CompilerParams(
            dimension_semantics=("parallel",),
            vmem_limit_bytes=40 * 1024 * 1024),
    )(bounds, rows_packed, cols3d, x, wrb, deg_col, y_i32)

    return out


# speedup vs baseline: 7.9157x; 1.2217x over previous
import functools

import jax
import jax.numpy as jnp
from jax.experimental import pallas as pl
from jax.experimental.pallas import tpu as pltpu


def _deg_kernel(cols_ref, deg_ref, *, u, nhi):
    k = pl.program_id(0)

    @pl.when(k == 0)
    def _init():
        deg_ref[...] = jnp.zeros_like(deg_ref)

    cvec = cols_ref[0, 0, :].reshape(1, u)
    lo_iota = jax.lax.broadcasted_iota(jnp.int32, (128, u), 0)
    hi_iota = jax.lax.broadcasted_iota(jnp.int32, (nhi, u), 0)
    xlo = (lo_iota == (cvec & 127)).astype(jnp.bfloat16)
    xhi = (hi_iota == (cvec >> 7)).astype(jnp.bfloat16)
    deg_ref[...] += jax.lax.dot_general(
        xlo, xhi, (((1,), (1,)), ((), ())),
        preferred_element_type=jnp.float32)


def _y_kernel(x_ref, w_ref, deg_ref, y_ref):
    xb = x_ref[...].astype(jnp.bfloat16)
    y = jnp.dot(xb, w_ref[...], preferred_element_type=jnp.float32)
    deg = deg_ref[...]
    rd = jnp.where(deg > 0.0, jax.lax.rsqrt(deg), 0.0)
    y_ref[...] = (y * rd).astype(jnp.bfloat16)


def _agg_kernel(bounds_ref, rows_ref, cols_ref, x_ref, wr_ref, deg_ref,
                y_ref, o_ref, tile_ref, *, tm, u, f_out):
    i = pl.program_id(0)
    c0 = i * tm
    k_lo = bounds_ref[i] // u
    k_hi = (bounds_ref[i + 1] + u - 1) // u

    o_ref[...] = jnp.zeros_like(o_ref)
    col_iota = c0 + jax.lax.broadcasted_iota(jnp.int32, (tm, u), 0)

    def body(k, carry):
        e0 = k * u
        for mi in range(u):
            q = pl.multiple_of(rows_ref[e0 + mi], 8)
            chunk = y_ref[pl.ds(q, 8), :]
            j = mi & 7
            tile_ref[pl.ds(mi, 1), :] = chunk[j:j + 1, :]

        cvec = cols_ref[k, 0, :].reshape(1, u)
        pt = (col_iota == cvec).astype(jnp.bfloat16)

        t = tile_ref[...]
        even = jax.lax.bitcast_convert_type(t << 16, jnp.float32)
        odd = jax.lax.bitcast_convert_type(t & jnp.int32(-65536), jnp.float32)
        eo = jnp.concatenate(
            [even.astype(jnp.bfloat16), odd.astype(jnp.bfloat16)], axis=1)

        o_ref[...] += jnp.dot(pt, eo, preferred_element_type=jnp.float32)
        return carry

    jax.lax.fori_loop(k_lo, k_hi, body, None)

    deg = deg_ref[...]
    rdc = jnp.where(deg > 0.0, jax.lax.rsqrt(deg), 0.0)
    xb = x_ref[...].astype(jnp.bfloat16)
    xwr = jnp.dot(xb, wr_ref[...], preferred_element_type=jnp.float32)
    o_ref[...] = xwr + rdc * o_ref[...]


def kernel(x, edge_index, weight, weight_r):
    N, f_in = x.shape
    f_out = weight.shape[1]
    E = edge_index.shape[1]

    TM = 256
    U = 512
    NBLK = N // TM
    half = f_out // 2

    row = edge_index[0]
    col = edge_index[1]
    shift = max(int(N - 1).bit_length(), 1)
    key = (col << shift) | row
    e_pad = ((E + U - 1) // U) * U
    if e_pad != E:
        key = jnp.concatenate(
            [key, jnp.full((e_pad - E,), N << shift, jnp.int32)])
    skey = jax.lax.sort(key, is_stable=False)
    col_s = skey >> shift
    row_s = skey & (N - 1)
    jpos = jnp.arange(e_pad, dtype=jnp.int32) & 7
    cshift = (jpos - row_s) & 7
    rows_q = cshift * (N + 8) + row_s - jpos + cshift
    cols3d = col_s.reshape(e_pad // U, 1, U)

    q = jnp.stack([jnp.arange(half, dtype=jnp.int32),
                   half + jnp.arange(half, dtype=jnp.int32)],
                  axis=1).reshape(f_out)
    wq = weight.astype(jnp.bfloat16)[:, q]
    wrb = weight_r.astype(jnp.bfloat16)

    degT = pl.pallas_call(
        functools.partial(_deg_kernel, u=U, nhi=N // 128),
        out_shape=jax.ShapeDtypeStruct((128, N // 128), jnp.float32),
        grid=(e_pad // U,),
        in_specs=[
            pl.BlockSpec((1, 1, U), lambda k: (k, 0, 0)),
        ],
        out_specs=pl.BlockSpec((128, N // 128), lambda k: (0, 0)),
        compiler_params=pltpu.CompilerParams(
            dimension_semantics=("arbitrary",),
            vmem_limit_bytes=40 * 1024 * 1024),
    )(cols3d)

    deg_lin = degT.T.reshape(N)
    deg_col = deg_lin[:, None]
    bounds = jnp.concatenate([
        jnp.zeros((1,), jnp.int32),
        jnp.cumsum(deg_lin.reshape(NBLK, TM).sum(axis=1)).astype(jnp.int32),
    ])

    TY = min(1024, N)
    yt = pl.pallas_call(
        _y_kernel,
        out_shape=jax.ShapeDtypeStruct((N, f_out), jnp.bfloat16),
        grid=(N // TY,),
        in_specs=[
            pl.BlockSpec((TY, f_in), lambda i: (i, 0)),
            pl.BlockSpec((f_in, f_out), lambda i: (0, 0)),
            pl.BlockSpec((TY, 1), lambda i: (i, 0)),
        ],
        out_specs=pl.BlockSpec((TY, f_out), lambda i: (i, 0)),
        compiler_params=pltpu.CompilerParams(
            dimension_semantics=("parallel",),
            vmem_limit_bytes=40 * 1024 * 1024),
    )(x, wq, deg_col)

    y_i32 = jax.lax.bitcast_convert_type(
        yt.reshape(N, half, 2), jnp.int32)
    y8 = jnp.concatenate(
        [jnp.pad(y_i32, ((c, 8 - c), (0, 0))) for c in range(8)], axis=0)

    out = pl.pallas_call(
        functools.partial(_agg_kernel, tm=TM, u=U, f_out=f_out),
        out_shape=jax.ShapeDtypeStruct((N, f_out), jnp.float32),
        grid_spec=pltpu.PrefetchScalarGridSpec(
            num_scalar_prefetch=2,
            grid=(NBLK,),
            in_specs=[
                pl.BlockSpec((e_pad // U, 1, U), lambda i, *_: (0, 0, 0)),
                pl.BlockSpec((TM, f_in), lambda i, *_: (i, 0)),
                pl.BlockSpec((f_in, f_out), lambda i, *_: (0, 0)),
                pl.BlockSpec((TM, 1), lambda i, *_: (i, 0)),
                pl.BlockSpec((8 * (N + 8), half), lambda i, *_: (0, 0)),
            ],
            out_specs=pl.BlockSpec((TM, f_out), lambda i, *_: (i, 0)),
            scratch_shapes=[
                pltpu.VMEM((U, half), jnp.int32),
            ],
        ),
        compiler_params=pltpu.CompilerParams(
            dimension_semantics=("parallel",),
            vmem_limit_bytes=56 * 1024 * 1024),
    )(bounds, rows_q, cols3d, x, wrb, deg_col, y8)

    return out


# speedup vs baseline: 8.1900x; 1.0347x over previous
import functools

import jax
import jax.numpy as jnp
from jax.experimental import pallas as pl
from jax.experimental.pallas import tpu as pltpu


def _deg_kernel(cols_ref, deg_ref, *, u, nhi):
    k = pl.program_id(0)

    @pl.when(k == 0)
    def _init():
        deg_ref[...] = jnp.zeros_like(deg_ref)

    cvec = cols_ref[0, 0, :].reshape(1, u)
    lo_iota = jax.lax.broadcasted_iota(jnp.int32, (128, u), 0)
    hi_iota = jax.lax.broadcasted_iota(jnp.int32, (nhi, u), 0)
    xlo = (lo_iota == (cvec & 127)).astype(jnp.bfloat16)
    xhi = (hi_iota == (cvec >> 7)).astype(jnp.bfloat16)
    deg_ref[...] += jax.lax.dot_general(
        xlo, xhi, (((1,), (1,)), ((), ())),
        preferred_element_type=jnp.float32)


def _y_kernel(x_ref, w_ref, deg_ref, y_ref):
    xb = x_ref[...].astype(jnp.bfloat16)
    y = jnp.dot(xb, w_ref[...], preferred_element_type=jnp.float32)
    deg = deg_ref[...]
    rd = jnp.where(deg > 0.0, jax.lax.rsqrt(deg), 0.0)
    y_ref[...] = (y * rd).astype(jnp.bfloat16)


def _agg_kernel(bounds_ref, rows_ref, cols_ref, x_ref, wr_ref, deg_ref,
                y_ref, o_ref, tile_ref, *, tm, u, f_out):
    i = pl.program_id(0)
    c0 = i * tm
    k_lo = bounds_ref[i] // u
    k_hi = (bounds_ref[i + 1] + u - 1) // u

    o_ref[...] = jnp.zeros_like(o_ref)
    col_iota = c0 + jax.lax.broadcasted_iota(jnp.int32, (tm, u), 0)

    def gather(k, par):
        e0 = k * u
        for mi in range(u):
            q = pl.multiple_of(rows_ref[e0 + mi], 8)
            chunk = y_ref[pl.ds(q, 8), :]
            j = mi & 7
            tile_ref[par, pl.ds(mi, 1), :] = chunk[j:j + 1, :]

    def consume(k, par):
        cvec = cols_ref[k, 0, :].reshape(1, u)
        pt = (col_iota == cvec).astype(jnp.bfloat16)

        t = tile_ref[par]
        even = jax.lax.bitcast_convert_type(t << 16, jnp.float32)
        odd = jax.lax.bitcast_convert_type(t & jnp.int32(-65536), jnp.float32)
        eo = jnp.concatenate(
            [even.astype(jnp.bfloat16), odd.astype(jnp.bfloat16)], axis=1)

        o_ref[...] += jnp.dot(pt, eo, preferred_element_type=jnp.float32)

    @pl.when(k_lo < k_hi)
    def _prologue():
        gather(k_lo, k_lo & 1)

    def body(k, carry):
        gather(k, k & 1)
        consume(k - 1, (k - 1) & 1)
        return carry

    jax.lax.fori_loop(k_lo + 1, k_hi, body, None)

    @pl.when(k_lo < k_hi)
    def _epilogue():
        consume(k_hi - 1, (k_hi - 1) & 1)

    deg = deg_ref[...]
    rdc = jnp.where(deg > 0.0, jax.lax.rsqrt(deg), 0.0)
    xb = x_ref[...].astype(jnp.bfloat16)
    xwr = jnp.dot(xb, wr_ref[...], preferred_element_type=jnp.float32)
    o_ref[...] = xwr + rdc * o_ref[...]


def kernel(x, edge_index, weight, weight_r):
    N, f_in = x.shape
    f_out = weight.shape[1]
    E = edge_index.shape[1]

    TM = 256
    U = 512
    NBLK = N // TM
    half = f_out // 2

    row = edge_index[0]
    col = edge_index[1]
    shift = max(int(N - 1).bit_length(), 1)
    key = (col << shift) | row
    e_pad = ((E + U - 1) // U) * U
    if e_pad != E:
        key = jnp.concatenate(
            [key, jnp.full((e_pad - E,), N << shift, jnp.int32)])
    skey = jax.lax.sort(key, is_stable=False)
    col_s = skey >> shift
    row_s = skey & (N - 1)
    jpos = jnp.arange(e_pad, dtype=jnp.int32) & 7
    cshift = (jpos - row_s) & 7
    rows_q = cshift * (N + 8) + row_s - jpos + cshift
    cols3d = col_s.reshape(e_pad // U, 1, U)

    q = jnp.stack([jnp.arange(half, dtype=jnp.int32),
                   half + jnp.arange(half, dtype=jnp.int32)],
                  axis=1).reshape(f_out)
    wq = weight.astype(jnp.bfloat16)[:, q]
    wrb = weight_r.astype(jnp.bfloat16)

    degT = pl.pallas_call(
        functools.partial(_deg_kernel, u=U, nhi=N // 128),
        out_shape=jax.ShapeDtypeStruct((128, N // 128), jnp.float32),
        grid=(e_pad // U,),
        in_specs=[
            pl.BlockSpec((1, 1, U), lambda k: (k, 0, 0)),
        ],
        out_specs=pl.BlockSpec((128, N // 128), lambda k: (0, 0)),
        compiler_params=pltpu.CompilerParams(
            dimension_semantics=("arbitrary",),
            vmem_limit_bytes=40 * 1024 * 1024),
    )(cols3d)

    deg_lin = degT.T.reshape(N)
    deg_col = deg_lin[:, None]
    bounds = jnp.concatenate([
        jnp.zeros((1,), jnp.int32),
        jnp.cumsum(deg_lin.reshape(NBLK, TM).sum(axis=1)).astype(jnp.int32),
    ])

    TY = min(1024, N)
    yt = pl.pallas_call(
        _y_kernel,
        out_shape=jax.ShapeDtypeStruct((N, f_out), jnp.bfloat16),
        grid=(N // TY,),
        in_specs=[
            pl.BlockSpec((TY, f_in), lambda i: (i, 0)),
            pl.BlockSpec((f_in, f_out), lambda i: (0, 0)),
            pl.BlockSpec((TY, 1), lambda i: (i, 0)),
        ],
        out_specs=pl.BlockSpec((TY, f_out), lambda i: (i, 0)),
        compiler_params=pltpu.CompilerParams(
            dimension_semantics=("parallel",),
            vmem_limit_bytes=40 * 1024 * 1024),
    )(x, wq, deg_col)

    y_i32 = jax.lax.bitcast_convert_type(
        yt.reshape(N, half, 2), jnp.int32)
    y8 = jnp.concatenate(
        [jnp.pad(y_i32, ((c, 8 - c), (0, 0))) for c in range(8)], axis=0)

    out = pl.pallas_call(
        functools.partial(_agg_kernel, tm=TM, u=U, f_out=f_out),
        out_shape=jax.ShapeDtypeStruct((N, f_out), jnp.float32),
        grid_spec=pltpu.PrefetchScalarGridSpec(
            num_scalar_prefetch=2,
            grid=(NBLK,),
            in_specs=[
                pl.BlockSpec((e_pad // U, 1, U), lambda i, *_: (0, 0, 0)),
                pl.BlockSpec((TM, f_in), lambda i, *_: (i, 0)),
                pl.BlockSpec((f_in, f_out), lambda i, *_: (0, 0)),
                pl.BlockSpec((TM, 1), lambda i, *_: (i, 0)),
                pl.BlockSpec((8 * (N + 8), half), lambda i, *_: (0, 0)),
            ],
            out_specs=pl.BlockSpec((TM, f_out), lambda i, *_: (i, 0)),
            scratch_shapes=[
                pltpu.VMEM((2, U, half), jnp.int32),
            ],
        ),
        compiler_params=pltpu.CompilerParams(
            dimension_semantics=("parallel",),
            vmem_limit_bytes=56 * 1024 * 1024),
    )(bounds, rows_q, cols3d, x, wrb, deg_col, y8)

    return out


# speedup vs baseline: 8.7116x; 1.0637x over previous
import functools

import jax
import jax.numpy as jnp
from jax.experimental import pallas as pl
from jax.experimental.pallas import tpu as pltpu


def _deg_kernel(cols_ref, deg_ref, *, u, nhi):
    k = pl.program_id(0)

    @pl.when(k == 0)
    def _init():
        deg_ref[...] = jnp.zeros_like(deg_ref)

    cvec = cols_ref[0, 0, :].reshape(1, u)
    lo_iota = jax.lax.broadcasted_iota(jnp.int32, (128, u), 0)
    hi_iota = jax.lax.broadcasted_iota(jnp.int32, (nhi, u), 0)
    xlo = (lo_iota == (cvec & 127)).astype(jnp.bfloat16)
    xhi = (hi_iota == (cvec >> 7)).astype(jnp.bfloat16)
    deg_ref[...] += jax.lax.dot_general(
        xlo, xhi, (((1,), (1,)), ((), ())),
        preferred_element_type=jnp.float32)


def _y_kernel(x_ref, w_ref, deg_ref, y_ref):
    xb = x_ref[...].astype(jnp.bfloat16)
    y = jnp.dot(xb, w_ref[...], preferred_element_type=jnp.float32)
    deg = deg_ref[...]
    rd = jnp.where(deg > 0.0, jax.lax.rsqrt(deg), 0.0)
    y_ref[...] = (y * rd).astype(jnp.bfloat16)


def _agg_kernel(bounds_ref, rows_ref, cols_ref, x_ref, wr_ref, deg_ref,
                y_ref, o_ref, tile_ref, *, tm, u, f_out):
    i = pl.program_id(0)
    c0 = i * tm
    k_lo = bounds_ref[i] // u
    k_hi = (bounds_ref[i + 1] + u - 1) // u

    o_ref[...] = jnp.zeros_like(o_ref)
    col_iota = c0 + jax.lax.broadcasted_iota(jnp.int32, (tm, u), 0)

    def gather(k, par):
        e0 = k * u
        for mi in range(u):
            q = pl.multiple_of(rows_ref[e0 + mi], 8)
            chunk = y_ref[pl.ds(q, 8), :]
            j = mi & 7
            tile_ref[par, pl.ds(mi, 1), :] = chunk[j:j + 1, :]

    def consume(k, par):
        cvec = cols_ref[k, 0, :].reshape(1, u)
        pt = (col_iota == cvec).astype(jnp.bfloat16)

        t = tile_ref[par]
        even = jax.lax.bitcast_convert_type(t << 16, jnp.float32)
        odd = jax.lax.bitcast_convert_type(t & jnp.int32(-65536), jnp.float32)
        eo = jnp.concatenate(
            [even.astype(jnp.bfloat16), odd.astype(jnp.bfloat16)], axis=1)

        o_ref[...] += jnp.dot(pt, eo, preferred_element_type=jnp.float32)

    @pl.when(k_lo < k_hi)
    def _prologue():
        gather(k_lo, k_lo & 1)

    def body(k, carry):
        gather(k, k & 1)
        consume(k - 1, (k - 1) & 1)
        return carry

    jax.lax.fori_loop(k_lo + 1, k_hi, body, None)

    @pl.when(k_lo < k_hi)
    def _epilogue():
        consume(k_hi - 1, (k_hi - 1) & 1)

    deg = deg_ref[...]
    rdc = jnp.where(deg > 0.0, jax.lax.rsqrt(deg), 0.0)
    xb = x_ref[...].astype(jnp.bfloat16)
    xwr = jnp.dot(xb, wr_ref[...], preferred_element_type=jnp.float32)
    o_ref[...] = xwr + rdc * o_ref[...]


def kernel(x, edge_index, weight, weight_r):
    N, f_in = x.shape
    f_out = weight.shape[1]
    E = edge_index.shape[1]

    TM = 256
    U = 512
    NBLK = N // TM
    half = f_out // 2

    row = edge_index[0]
    col = edge_index[1]
    shift = max(int(N - 1).bit_length(), 1)
    key = (col << shift) | row
    e_pad = ((E + U - 1) // U) * U
    if e_pad != E:
        key = jnp.concatenate(
            [key, jnp.full((e_pad - E,), N << shift, jnp.int32)])
    skey = jax.lax.sort(key, is_stable=False)
    col_s = skey >> shift
    row_s = skey & (N - 1)
    jpos = jnp.arange(e_pad, dtype=jnp.int32) & 7
    cshift = (jpos - row_s) & 7
    rows_q = cshift * (N + 8) + row_s - jpos + cshift
    cols3d = col_s.reshape(e_pad // U, 1, U)

    q = jnp.stack([jnp.arange(half, dtype=jnp.int32),
                   half + jnp.arange(half, dtype=jnp.int32)],
                  axis=1).reshape(f_out)
    wq = weight.astype(jnp.bfloat16)[:, q]
    wrb = weight_r.astype(jnp.bfloat16)

    UD = 2048 if e_pad % 2048 == 0 else U
    degT = pl.pallas_call(
        functools.partial(_deg_kernel, u=UD, nhi=N // 128),
        out_shape=jax.ShapeDtypeStruct((128, N // 128), jnp.float32),
        grid=(e_pad // UD,),
        in_specs=[
            pl.BlockSpec((1, 1, UD), lambda k: (k, 0, 0)),
        ],
        out_specs=pl.BlockSpec((128, N // 128), lambda k: (0, 0)),
        compiler_params=pltpu.CompilerParams(
            dimension_semantics=("arbitrary",),
            vmem_limit_bytes=40 * 1024 * 1024),
    )(col_s.reshape(e_pad // UD, 1, UD))

    deg_lin = degT.T.reshape(N)
    deg_col = deg_lin[:, None]
    bounds = jnp.concatenate([
        jnp.zeros((1,), jnp.int32),
        jnp.cumsum(deg_lin.reshape(NBLK, TM).sum(axis=1)).astype(jnp.int32),
    ])

    TY = min(1024, N)
    yt = pl.pallas_call(
        _y_kernel,
        out_shape=jax.ShapeDtypeStruct((N, f_out), jnp.bfloat16),
        grid=(N // TY,),
        in_specs=[
            pl.BlockSpec((TY, f_in), lambda i: (i, 0)),
            pl.BlockSpec((f_in, f_out), lambda i: (0, 0)),
            pl.BlockSpec((TY, 1), lambda i: (i, 0)),
        ],
        out_specs=pl.BlockSpec((TY, f_out), lambda i: (i, 0)),
        compiler_params=pltpu.CompilerParams(
            dimension_semantics=("parallel",),
            vmem_limit_bytes=40 * 1024 * 1024),
    )(x, wq, deg_col)

    y_i32 = jax.lax.bitcast_convert_type(
        yt.reshape(N, half, 2), jnp.int32)
    y8 = jnp.concatenate(
        [jnp.pad(y_i32, ((c, 8 - c), (0, 0))) for c in range(8)], axis=0)

    out = pl.pallas_call(
        functools.partial(_agg_kernel, tm=TM, u=U, f_out=f_out),
        out_shape=jax.ShapeDtypeStruct((N, f_out), jnp.float32),
        grid_spec=pltpu.PrefetchScalarGridSpec(
            num_scalar_prefetch=2,
            grid=(NBLK,),
            in_specs=[
                pl.BlockSpec((e_pad // U, 1, U), lambda i, *_: (0, 0, 0)),
                pl.BlockSpec((TM, f_in), lambda i, *_: (i, 0)),
                pl.BlockSpec((f_in, f_out), lambda i, *_: (0, 0)),
                pl.BlockSpec((TM, 1), lambda i, *_: (i, 0)),
                pl.BlockSpec((8 * (N + 8), half), lambda i, *_: (0, 0)),
            ],
            out_specs=pl.BlockSpec((TM, f_out), lambda i, *_: (i, 0)),
            scratch_shapes=[
                pltpu.VMEM((2, U, half), jnp.int32),
            ],
        ),
        compiler_params=pltpu.CompilerParams(
            dimension_semantics=("parallel",),
            vmem_limit_bytes=56 * 1024 * 1024),
    )(bounds, rows_q, cols3d, x, wrb, deg_col, y8)

    return out
